# baseline (speedup 1.0000x reference)
import math

import jax
import jax.numpy as jnp
from jax import lax
from jax.experimental import pallas as pl
from jax.experimental.pallas import tpu as pltpu
from jax.experimental.pallas import tpu_sc as plsc

B, C_IN, T, C_LAT, K = 64, 128, 256, 64, 512
BB = 16
W_IN = 16
W_OUT = 8
SW = 8
_LN_EMA = math.log(0.001)
_BF = jnp.bfloat16
_DN_BT = (((1,), (1,)), ((), ()))
_DN_NN = (((1,), (0,)), ((), ()))
_DN_TN = (((0,), (0,)), ((), ()))


def _enc_body(wt_ref, x_ref, y_ref):
    wb = wt_ref[...].astype(_BF)
    for i in range(BB):
        y_ref[i] = lax.dot_general(wb, x_ref[i].astype(_BF), _DN_NN,
                                   preferred_element_type=jnp.float32)


def _sc_copy_body(m_ref, o_ref, sem):
    idx = lax.axis_index("core")
    h = K // 2
    pltpu.async_copy(m_ref.at[pl.ds(idx * h, h)],
                     o_ref.at[pl.ds(idx * h, h)], sem).wait()


def _mega_body(m_hbm, y_hbm, val_hbm, z_ref, mscr, yscr, sT, cwb, gb, obuf,
               msem, ysem, osem):
    def start_in(j):
        pltpu.make_async_copy(m_hbm.at[:, j, :], mscr.at[j],
                              msem.at[j % W_IN]).start()
        pltpu.make_async_copy(y_hbm.at[:, j, :], yscr.at[j],
                              ysem.at[j % W_IN]).start()

    def wait_in(j):
        pltpu.make_async_copy(m_hbm.at[:, j, :], mscr.at[j],
                              msem.at[j % W_IN]).wait()
        pltpu.make_async_copy(y_hbm.at[:, j, :], yscr.at[j],
                              ysem.at[j % W_IN]).wait()

    for jj in range(W_IN):
        start_in(jj)

    def phase_a(j, carry):
        wait_in(j)

        @pl.when(j < C_LAT - W_IN)
        def _():
            start_in(j + W_IN)

        mj = mscr[j]
        dotT = lax.dot_general(mj.astype(_BF), yscr[j].astype(_BF), _DN_BT,
                               preferred_element_type=jnp.float32)
        inc = jnp.sum(mj * mj, axis=1, keepdims=True) - 2.0 * dotT

        @pl.when(j == 0)
        def _():
            sT[...] = inc

        @pl.when(j > 0)
        def _():
            sT[...] += inc

        return carry

    lax.fori_loop(0, C_LAT, phase_a, 0)

    s = sT[...]
    minv = jnp.min(s, axis=0, keepdims=True)
    kio = lax.broadcasted_iota(jnp.int32, (K, B), 0)
    z = jnp.min(jnp.where(s == minv, kio, K), axis=0, keepdims=True)
    z_ref[...] = z
    oh = (kio == z)
    ohb = oh.astype(_BF)
    eq = lax.dot_general(ohb, ohb, _DN_TN,
                         preferred_element_type=jnp.float32)
    jio = lax.broadcasted_iota(jnp.int32, (B, B), 0)
    iio = lax.broadcasted_iota(jnp.int32, (B, B), 1)
    later = jnp.sum(jnp.where(jio > iio, eq, 0.0), axis=0, keepdims=True)
    w = 0.999 * jnp.exp(later * _LN_EMA)
    count = jnp.sum(oh.astype(jnp.float32), axis=1, keepdims=True)
    scale = jnp.exp(count * _LN_EMA)
    cwb[...] = (oh.astype(jnp.float32) * scale).astype(_BF)
    gb[...] = (eq * w).astype(_BF)

    def phase_c(j, carry):
        @pl.when(j >= W_OUT)
        def _():
            pltpu.make_async_copy(obuf.at[j % W_OUT],
                                  val_hbm.at[j - W_OUT],
                                  osem.at[j % W_OUT]).wait()

        val_m = lax.dot_general(cwb[...], mscr[j].astype(_BF), _DN_TN,
                                preferred_element_type=jnp.float32)
        val_y = lax.dot_general(gb[...], yscr[j].astype(_BF), _DN_NN,
                                preferred_element_type=jnp.float32)
        obuf[j % W_OUT] = val_m + val_y
        pltpu.make_async_copy(obuf.at[j % W_OUT], val_hbm.at[j],
                              osem.at[j % W_OUT]).start()
        return carry

    lax.fori_loop(0, C_LAT, phase_c, 0)
    for jj in range(C_LAT - W_OUT, C_LAT):
        pltpu.make_async_copy(obuf.at[jj % W_OUT], val_hbm.at[jj],
                              osem.at[jj % W_OUT]).wait()


def _scatter_body(z_any, val_any, src_any, out_any, zsm, zsem, sem):
    del src_any
    cp = pltpu.make_async_copy(z_any, zsm, zsem)
    cp.start()
    cp.wait()

    def issue(i):
        pltpu.make_async_copy(val_any.at[:, i, :], out_any.at[zsm[0, i]],
                              sem.at[i % SW]).start()

    for i in range(SW):
        issue(i)

    def loop(i, carry):
        pltpu.make_async_copy(val_any.at[:, i - SW, :],
                              out_any.at[zsm[0, i - SW]],
                              sem.at[(i - SW) % SW]).wait()
        issue(i)
        return carry

    lax.fori_loop(SW, B, loop, 0)
    for i in range(B - SW, B):
        pltpu.make_async_copy(val_any.at[:, i, :], out_any.at[zsm[0, i]],
                              sem.at[i % SW]).wait()


def kernel(x, W_enc, m, sd, p):
    del sd, p

    sc_copy = pl.kernel(
        _sc_copy_body,
        out_type=jax.ShapeDtypeStruct((K, C_LAT, T), jnp.float32),
        mesh=plsc.ScalarSubcoreMesh(axis_name="core", num_cores=2),
        scratch_types=[pltpu.SemaphoreType.DMA],
    )
    out_base = sc_copy(m)

    y = pl.pallas_call(
        _enc_body,
        grid=(B // BB,),
        in_specs=[pl.BlockSpec((C_LAT, C_IN), lambda i: (0, 0)),
                  pl.BlockSpec((BB, C_IN, T), lambda i: (i, 0, 0))],
        out_specs=pl.BlockSpec((BB, C_LAT, T), lambda i: (i, 0, 0)),
        out_shape=jax.ShapeDtypeStruct((B, C_LAT, T), jnp.float32),
    )(W_enc.T, x)

    any_spec = pl.BlockSpec(memory_space=pl.ANY)
    val, z = pl.pallas_call(
        _mega_body,
        in_specs=[any_spec, any_spec],
        out_specs=[any_spec, pl.BlockSpec((1, B), lambda: (0, 0))],
        out_shape=[jax.ShapeDtypeStruct((C_LAT, B, T), jnp.float32),
                   jax.ShapeDtypeStruct((1, B), jnp.int32)],
        scratch_shapes=[
            pltpu.VMEM((C_LAT, K, T), jnp.float32),
            pltpu.VMEM((C_LAT, B, T), jnp.float32),
            pltpu.VMEM((K, B), jnp.float32),
            pltpu.VMEM((K, B), _BF),
            pltpu.VMEM((B, B), _BF),
            pltpu.VMEM((W_OUT, B, T), jnp.float32),
            pltpu.SemaphoreType.DMA((W_IN,)),
            pltpu.SemaphoreType.DMA((W_IN,)),
            pltpu.SemaphoreType.DMA((W_OUT,)),
        ],
    )(m, y)

    out = pl.pallas_call(
        _scatter_body,
        in_specs=[any_spec, any_spec, any_spec],
        out_specs=any_spec,
        out_shape=jax.ShapeDtypeStruct((K, C_LAT, T), jnp.float32),
        scratch_shapes=[pltpu.SMEM((1, B), jnp.int32),
                        pltpu.SemaphoreType.DMA,
                        pltpu.SemaphoreType.DMA((SW,))],
        input_output_aliases={2: 0},
    )(z, val, out_base)

    return out


# speedup vs baseline: 27.2803x; 27.2803x over previous
import math

import jax
import jax.numpy as jnp
from jax import lax
from jax.experimental import pallas as pl
from jax.experimental.pallas import tpu as pltpu

B, C_IN, T, C_LAT, K = 64, 128, 256, 64, 512
BB = 16
W_IN = 16
W_OUT = 8
NS = 2
KH = K // NS
_LN_EMA = math.log(0.001)
_BF = jnp.bfloat16
_DN_BT = (((1,), (1,)), ((), ()))
_DN_NN = (((1,), (0,)), ((), ()))


def _enc_body(wt_ref, x_ref, y_ref):
    wb = wt_ref[...].astype(_BF)
    for i in range(BB):
        y_ref[i] = lax.dot_general(wb, x_ref[i].astype(_BF), _DN_NN,
                                   preferred_element_type=jnp.float32)


def _mega_body(m_hbm, y_hbm, out_hbm, mscr, yscr, sT, cw, scl, obuf,
               msem, ysem, osem):
    def start_in(j):
        for h in range(NS):
            pltpu.make_async_copy(m_hbm.at[pl.ds(h * KH, KH), j, :],
                                  mscr.at[j, pl.ds(h * KH, KH)],
                                  msem.at[j % W_IN, h]).start()
        pltpu.make_async_copy(y_hbm.at[:, j, :], yscr.at[j],
                              ysem.at[j % W_IN]).start()

    def wait_in(j):
        for h in range(NS):
            pltpu.make_async_copy(m_hbm.at[pl.ds(h * KH, KH), j, :],
                                  mscr.at[j, pl.ds(h * KH, KH)],
                                  msem.at[j % W_IN, h]).wait()
        pltpu.make_async_copy(y_hbm.at[:, j, :], yscr.at[j],
                              ysem.at[j % W_IN]).wait()

    for jj in range(W_IN):
        start_in(jj)

    def phase_a(j, carry):
        wait_in(j)

        @pl.when(j < C_LAT - W_IN)
        def _():
            start_in(j + W_IN)

        mj = mscr[j]
        dotT = lax.dot_general(mj.astype(_BF), yscr[j].astype(_BF), _DN_BT,
                               preferred_element_type=jnp.float32)
        inc = jnp.sum(mj * mj, axis=1, keepdims=True) - 2.0 * dotT

        @pl.when(j == 0)
        def _():
            sT[...] = inc

        @pl.when(j > 0)
        def _():
            sT[...] += inc

        return carry

    lax.fori_loop(0, C_LAT, phase_a, 0)

    s = sT[...]
    minv = jnp.min(s, axis=0, keepdims=True)
    kio = lax.broadcasted_iota(jnp.int32, (K, B), 0)
    z = jnp.min(jnp.where(s == minv, kio, K), axis=0, keepdims=True)
    oh = (kio == z)
    ohb = oh.astype(_BF)
    eq = lax.dot_general(ohb, ohb, (((0,), (0,)), ((), ())),
                         preferred_element_type=jnp.float32)
    jio = lax.broadcasted_iota(jnp.int32, (B, B), 0)
    iio = lax.broadcasted_iota(jnp.int32, (B, B), 1)
    later = jnp.sum(jnp.where(jio > iio, eq, 0.0), axis=0, keepdims=True)
    w = 0.999 * jnp.exp(later * _LN_EMA)
    cw[...] = (oh.astype(jnp.float32) * w).astype(_BF)
    count = jnp.sum(oh.astype(jnp.float32), axis=1, keepdims=True)
    scl[...] = jnp.exp(count * _LN_EMA)

    def start_out(j):
        for h in range(NS):
            pltpu.make_async_copy(obuf.at[j % W_OUT, pl.ds(h * KH, KH)],
                                  out_hbm.at[pl.ds(h * KH, KH), j, :],
                                  osem.at[j % W_OUT, h]).start()

    def wait_out(j):
        for h in range(NS):
            pltpu.make_async_copy(obuf.at[j % W_OUT, pl.ds(h * KH, KH)],
                                  out_hbm.at[pl.ds(h * KH, KH), j, :],
                                  osem.at[j % W_OUT, h]).wait()

    def phase_c(j, carry):
        @pl.when(j >= W_OUT)
        def _():
            wait_out(j - W_OUT)

        upd = lax.dot_general(cw[...], yscr[j].astype(_BF), _DN_NN,
                              preferred_element_type=jnp.float32)
        obuf[j % W_OUT] = scl[...] * mscr[j] + upd
        start_out(j)
        return carry

    lax.fori_loop(0, C_LAT, phase_c, 0)
    for jj in range(C_LAT - W_OUT, C_LAT):
        wait_out(jj)


def kernel(x, W_enc, m, sd, p):
    del sd, p

    y = pl.pallas_call(
        _enc_body,
        grid=(B // BB,),
        in_specs=[pl.BlockSpec((C_LAT, C_IN), lambda i: (0, 0)),
                  pl.BlockSpec((BB, C_IN, T), lambda i: (i, 0, 0))],
        out_specs=pl.BlockSpec((BB, C_LAT, T), lambda i: (i, 0, 0)),
        out_shape=jax.ShapeDtypeStruct((B, C_LAT, T), jnp.float32),
    )(W_enc.T, x)

    any_spec = pl.BlockSpec(memory_space=pl.ANY)
    out = pl.pallas_call(
        _mega_body,
        in_specs=[any_spec, any_spec],
        out_specs=any_spec,
        out_shape=jax.ShapeDtypeStruct((K, C_LAT, T), jnp.float32),
        scratch_shapes=[
            pltpu.VMEM((C_LAT, K, T), jnp.float32),
            pltpu.VMEM((C_LAT, B, T), jnp.float32),
            pltpu.VMEM((K, B), jnp.float32),
            pltpu.VMEM((K, B), _BF),
            pltpu.VMEM((K, 1), jnp.float32),
            pltpu.VMEM((W_OUT, K, T), jnp.float32),
            pltpu.SemaphoreType.DMA((W_IN, NS)),
            pltpu.SemaphoreType.DMA((W_IN,)),
            pltpu.SemaphoreType.DMA((W_OUT, NS)),
        ],
    )(m, y)

    return out


# speedup vs baseline: 27.8489x; 1.0208x over previous
import math

import jax
import jax.numpy as jnp
from jax import lax
from jax.experimental import pallas as pl
from jax.experimental.pallas import tpu as pltpu

B, C_IN, T, C_LAT, K = 64, 128, 256, 64, 512
BB = 16
W_IN = 16
W_OUT = 8
_LN_EMA = math.log(0.001)
_BF = jnp.bfloat16
_DN_BT = (((1,), (1,)), ((), ()))
_DN_NN = (((1,), (0,)), ((), ()))


def _enc_body(wt_ref, x_ref, y_ref):
    wb = wt_ref[...].astype(_BF)
    for i in range(BB):
        y_ref[i] = lax.dot_general(wb, x_ref[i].astype(_BF), _DN_NN,
                                   preferred_element_type=jnp.float32)


def _mega_body(m_hbm, y_hbm, out_hbm, mscr, yscr, sT, cw, scl, obuf,
               msem, ysem, osem):
    def start_in(j):
        pltpu.make_async_copy(m_hbm.at[:, j, :], mscr.at[j],
                              msem.at[j % W_IN]).start()
        pltpu.make_async_copy(y_hbm.at[:, j, :], yscr.at[j],
                              ysem.at[j % W_IN]).start()

    def wait_in(j):
        pltpu.make_async_copy(m_hbm.at[:, j, :], mscr.at[j],
                              msem.at[j % W_IN]).wait()
        pltpu.make_async_copy(y_hbm.at[:, j, :], yscr.at[j],
                              ysem.at[j % W_IN]).wait()

    for jj in range(W_IN):
        start_in(jj)

    def phase_a(j, carry):
        wait_in(j)

        @pl.when(j < C_LAT - W_IN)
        def _():
            start_in(j + W_IN)

        mj = mscr[j]
        dotT = lax.dot_general(mj.astype(_BF), yscr[j].astype(_BF), _DN_BT,
                               preferred_element_type=jnp.float32)
        inc = jnp.sum(mj * mj, axis=1, keepdims=True) - 2.0 * dotT

        @pl.when(j == 0)
        def _():
            sT[...] = inc

        @pl.when(j > 0)
        def _():
            sT[...] += inc

        return carry

    lax.fori_loop(0, C_LAT, phase_a, 0)

    s = sT[...]
    minv = jnp.min(s, axis=0, keepdims=True)
    kio = lax.broadcasted_iota(jnp.int32, (K, B), 0)
    z = jnp.min(jnp.where(s == minv, kio, K), axis=0, keepdims=True)
    oh = (kio == z)
    ohb = oh.astype(_BF)
    eq = lax.dot_general(ohb, ohb, (((0,), (0,)), ((), ())),
                         preferred_element_type=jnp.float32)
    jio = lax.broadcasted_iota(jnp.int32, (B, B), 0)
    iio = lax.broadcasted_iota(jnp.int32, (B, B), 1)
    later = jnp.sum(jnp.where(jio > iio, eq, 0.0), axis=0, keepdims=True)
    w = 0.999 * jnp.exp(later * _LN_EMA)
    cw[...] = (oh.astype(jnp.float32) * w).astype(_BF)
    count = jnp.sum(oh.astype(jnp.float32), axis=1, keepdims=True)
    scl[...] = jnp.exp(count * _LN_EMA)

    def phase_c(j, carry):
        @pl.when(j >= W_OUT)
        def _():
            pltpu.make_async_copy(obuf.at[j % W_OUT],
                                  out_hbm.at[:, j - W_OUT, :],
                                  osem.at[j % W_OUT]).wait()

        upd = lax.dot_general(cw[...], yscr[j].astype(_BF), _DN_NN,
                              preferred_element_type=jnp.float32)
        obuf[j % W_OUT] = scl[...] * mscr[j] + upd
        pltpu.make_async_copy(obuf.at[j % W_OUT], out_hbm.at[:, j, :],
                              osem.at[j % W_OUT]).start()
        return carry

    lax.fori_loop(0, C_LAT, phase_c, 0)
    for jj in range(C_LAT - W_OUT, C_LAT):
        pltpu.make_async_copy(obuf.at[jj % W_OUT], out_hbm.at[:, jj, :],
                              osem.at[jj % W_OUT]).wait()


def kernel(x, W_enc, m, sd, p):
    del sd, p

    y = pl.pallas_call(
        _enc_body,
        grid=(B // BB,),
        in_specs=[pl.BlockSpec((C_LAT, C_IN), lambda i: (0, 0)),
                  pl.BlockSpec((BB, C_IN, T), lambda i: (i, 0, 0))],
        out_specs=pl.BlockSpec((BB, C_LAT, T), lambda i: (i, 0, 0)),
        out_shape=jax.ShapeDtypeStruct((B, C_LAT, T), jnp.float32),
    )(W_enc.T, x)

    any_spec = pl.BlockSpec(memory_space=pl.ANY)
    out = pl.pallas_call(
        _mega_body,
        in_specs=[any_spec, any_spec],
        out_specs=any_spec,
        out_shape=jax.ShapeDtypeStruct((K, C_LAT, T), jnp.float32),
        scratch_shapes=[
            pltpu.VMEM((C_LAT, K, T), jnp.float32),
            pltpu.VMEM((C_LAT, B, T), jnp.float32),
            pltpu.VMEM((K, B), jnp.float32),
            pltpu.VMEM((K, B), _BF),
            pltpu.VMEM((K, 1), jnp.float32),
            pltpu.VMEM((W_OUT, K, T), jnp.float32),
            pltpu.SemaphoreType.DMA((W_IN,)),
            pltpu.SemaphoreType.DMA((W_IN,)),
            pltpu.SemaphoreType.DMA((W_OUT,)),
        ],
    )(m, y)

    return out


# speedup vs baseline: 27.8847x; 1.0013x over previous
import math

import jax
import jax.numpy as jnp
from jax import lax
from jax.experimental import pallas as pl
from jax.experimental.pallas import tpu as pltpu

B, C_IN, T, C_LAT, K = 64, 128, 256, 64, 512
BB = 16
W_IN = 24
W_OUT = 16
_LN_EMA = math.log(0.001)
_BF = jnp.bfloat16
_DN_BT = (((1,), (1,)), ((), ()))
_DN_NN = (((1,), (0,)), ((), ()))


def _enc_body(wt_ref, x_ref, y_ref):
    wb = wt_ref[...].astype(_BF)
    for i in range(BB):
        y_ref[i] = lax.dot_general(wb, x_ref[i].astype(_BF), _DN_NN,
                                   preferred_element_type=jnp.float32)


def _mega_body(m_hbm, y_hbm, out_hbm, mscr, yscr, sT, cw, scl, obuf,
               msem, ysem, osem):
    def start_in(j):
        pltpu.make_async_copy(m_hbm.at[:, j, :], mscr.at[j],
                              msem.at[j % W_IN]).start()
        pltpu.make_async_copy(y_hbm.at[:, j, :], yscr.at[j],
                              ysem.at[j % W_IN]).start()

    def wait_in(j):
        pltpu.make_async_copy(m_hbm.at[:, j, :], mscr.at[j],
                              msem.at[j % W_IN]).wait()
        pltpu.make_async_copy(y_hbm.at[:, j, :], yscr.at[j],
                              ysem.at[j % W_IN]).wait()

    for jj in range(W_IN):
        start_in(jj)

    def phase_a(j, carry):
        wait_in(j)

        @pl.when(j < C_LAT - W_IN)
        def _():
            start_in(j + W_IN)

        mj = mscr[j]
        dotT = lax.dot_general(mj.astype(_BF), yscr[j].astype(_BF), _DN_BT,
                               preferred_element_type=jnp.float32)
        inc = jnp.sum(mj * mj, axis=1, keepdims=True) - 2.0 * dotT

        @pl.when(j == 0)
        def _():
            sT[...] = inc

        @pl.when(j > 0)
        def _():
            sT[...] += inc

        return carry

    lax.fori_loop(0, C_LAT, phase_a, 0)

    s = sT[...]
    minv = jnp.min(s, axis=0, keepdims=True)
    kio = lax.broadcasted_iota(jnp.int32, (K, B), 0)
    z = jnp.min(jnp.where(s == minv, kio, K), axis=0, keepdims=True)
    oh = (kio == z)
    ohb = oh.astype(_BF)
    eq = lax.dot_general(ohb, ohb, (((0,), (0,)), ((), ())),
                         preferred_element_type=jnp.float32)
    jio = lax.broadcasted_iota(jnp.int32, (B, B), 0)
    iio = lax.broadcasted_iota(jnp.int32, (B, B), 1)
    later = jnp.sum(jnp.where(jio > iio, eq, 0.0), axis=0, keepdims=True)
    w = 0.999 * jnp.exp(later * _LN_EMA)
    cw[...] = (oh.astype(jnp.float32) * w).astype(_BF)
    count = jnp.sum(oh.astype(jnp.float32), axis=1, keepdims=True)
    scl[...] = jnp.exp(count * _LN_EMA)

    def phase_c(j, carry):
        @pl.when(j >= W_OUT)
        def _():
            pltpu.make_async_copy(obuf.at[j % W_OUT],
                                  out_hbm.at[:, j - W_OUT, :],
                                  osem.at[j % W_OUT]).wait()

        upd = lax.dot_general(cw[...], yscr[j].astype(_BF), _DN_NN,
                              preferred_element_type=jnp.float32)
        obuf[j % W_OUT] = scl[...] * mscr[j] + upd
        pltpu.make_async_copy(obuf.at[j % W_OUT], out_hbm.at[:, j, :],
                              osem.at[j % W_OUT]).start()
        return carry

    lax.fori_loop(0, C_LAT, phase_c, 0)
    for jj in range(C_LAT - W_OUT, C_LAT):
        pltpu.make_async_copy(obuf.at[jj % W_OUT], out_hbm.at[:, jj, :],
                              osem.at[jj % W_OUT]).wait()


def kernel(x, W_enc, m, sd, p):
    del sd, p

    y = pl.pallas_call(
        _enc_body,
        grid=(B // BB,),
        in_specs=[pl.BlockSpec((C_LAT, C_IN), lambda i: (0, 0)),
                  pl.BlockSpec((BB, C_IN, T), lambda i: (i, 0, 0))],
        out_specs=pl.BlockSpec((BB, C_LAT, T), lambda i: (i, 0, 0)),
        out_shape=jax.ShapeDtypeStruct((B, C_LAT, T), jnp.float32),
    )(W_enc.T, x)

    any_spec = pl.BlockSpec(memory_space=pl.ANY)
    out = pl.pallas_call(
        _mega_body,
        in_specs=[any_spec, any_spec],
        out_specs=any_spec,
        out_shape=jax.ShapeDtypeStruct((K, C_LAT, T), jnp.float32),
        scratch_shapes=[
            pltpu.VMEM((C_LAT, K, T), jnp.float32),
            pltpu.VMEM((C_LAT, B, T), jnp.float32),
            pltpu.VMEM((K, B), jnp.float32),
            pltpu.VMEM((K, B), _BF),
            pltpu.VMEM((K, 1), jnp.float32),
            pltpu.VMEM((W_OUT, K, T), jnp.float32),
            pltpu.SemaphoreType.DMA((W_IN,)),
            pltpu.SemaphoreType.DMA((W_IN,)),
            pltpu.SemaphoreType.DMA((W_OUT,)),
        ],
    )(m, y)

    return out
